# baseline (speedup 1.0000x reference)
import functools

import jax
import jax.numpy as jnp
from jax.experimental import pallas as pl
from jax.experimental.pallas import tpu as pltpu

NC, NP, D, E = 32, 2048, 128, 128
PB = 256
NBLK = NP // PB
NEG = -1e30


def _elu(x):
    return jnp.where(x > 0, x, jnp.exp(x) - 1.0)


def _main_body(xc_ref, xp_ref, mT_ref,
               Wc0_ref, bc0_ref, Wc1_ref, bc1_ref,
               Wp0_ref, bp0_ref, Wp1_ref, bp1_ref,
               A0c_ref, A0p_ref, A0b_ref, A1_ref, A1b_ref, A2t_ref, A2b_ref,
               lg_ref, hps_ref):
    hc = _elu(jnp.dot(xc_ref[:, :], Wc0_ref[:, :],
                      preferred_element_type=jnp.float32) + bc0_ref[:, :])
    hc = _elu(jnp.dot(hc, Wc1_ref[:, :],
                      preferred_element_type=jnp.float32) + bc1_ref[:, :])
    U = jnp.dot(hc, A0c_ref[:, :],
                preferred_element_type=jnp.float32) + A0b_ref[:, :]

    hp = _elu(jnp.dot(xp_ref[:, :], Wp0_ref[:, :],
                      preferred_element_type=jnp.float32) + bp0_ref[:, :])
    hp = _elu(jnp.dot(hp, Wp1_ref[:, :],
                      preferred_element_type=jnp.float32) + bp1_ref[:, :])
    hps_ref[0, :, :] = jnp.sum(hp, axis=0, keepdims=True)

    V = jnp.dot(hp, A0p_ref[:, :], preferred_element_type=jnp.float32)
    ha = _elu(V[:, None, :] + U[None, :, :])
    ha = ha.reshape(PB * NC, 2 * E)
    ha = _elu(jnp.dot(ha, A1_ref[:, :],
                      preferred_element_type=jnp.float32) + A1b_ref[:, :])
    lg = jnp.sum(ha.reshape(PB, NC, 2 * E) * A2t_ref[:, :][None, :, :],
                 axis=-1) + A2b_ref[0, 0]
    lg_ref[:, :] = jnp.where(mT_ref[:, :] > 0, lg, NEG)


def _fin_body(lg_ref, hps_ref, xc_ref,
              Wc0_ref, bc0_ref, Wc1_ref, bc1_ref,
              C0_ref, C0b_ref, C1_ref, C1b_ref, C2t_ref, C2b_ref,
              act_ref, lp_ref, val_ref):
    full = lg_ref[:, :]
    M = jnp.max(full)
    S = jnp.sum(jnp.exp(full - M))
    pidx = jax.lax.broadcasted_iota(jnp.int32, (NP, NC), 0)
    cidx = jax.lax.broadcasted_iota(jnp.int32, (NP, NC), 1)
    flat = pidx * NC + cidx
    act_ref[0, 0] = jnp.min(jnp.where(full >= M, flat, jnp.int32(2147483647)))
    lp_ref[0, 0] = -jnp.log(S)
    hc = _elu(jnp.dot(xc_ref[:, :], Wc0_ref[:, :],
                      preferred_element_type=jnp.float32) + bc0_ref[:, :])
    hc = _elu(jnp.dot(hc, Wc1_ref[:, :],
                      preferred_element_type=jnp.float32) + bc1_ref[:, :])
    hpool = jnp.concatenate(
        [jnp.mean(hc, axis=0, keepdims=True),
         jnp.sum(hps_ref[:, :], axis=0, keepdims=True) * (1.0 / NP)],
        axis=1)
    hv = _elu(jnp.dot(hpool, C0_ref[:, :],
                      preferred_element_type=jnp.float32) + C0b_ref[:, :])
    hv = _elu(jnp.dot(hv, C1_ref[:, :],
                      preferred_element_type=jnp.float32) + C1b_ref[:, :])
    val_ref[0, 0] = jnp.sum(hv * C2t_ref[:, :]) + C2b_ref[0, 0]


@jax.jit
def _run(x_crane, x_pile, maskT, Wc0, bc0, Wc1, bc1, Wp0, bp0, Wp1, bp1,
         A0c, A0p, A0b, A1, A1b, A2t, A2b, C0, C0b, C1, C1b, C2t, C2b):
    full = lambda shape: pl.BlockSpec(shape, lambda i: (0,) * len(shape))
    lg, hps = pl.pallas_call(
        _main_body,
        grid=(NBLK,),
        in_specs=[
            full((NC, D)),
            pl.BlockSpec((PB, D), lambda i: (i, 0)),
            pl.BlockSpec((PB, NC), lambda i: (i, 0)),
            full((D, E)), full((1, E)), full((E, E)), full((1, E)),
            full((D, E)), full((1, E)), full((E, E)), full((1, E)),
            full((D, 2 * E)), full((D, 2 * E)), full((1, 2 * E)),
            full((2 * E, 2 * E)), full((1, 2 * E)),
            full((1, 2 * E)), full((1, 1)),
        ],
        out_specs=[
            pl.BlockSpec((PB, NC), lambda i: (i, 0)),
            pl.BlockSpec((1, 1, D), lambda i: (i, 0, 0)),
        ],
        out_shape=[
            jax.ShapeDtypeStruct((NP, NC), jnp.float32),
            jax.ShapeDtypeStruct((NBLK, 1, D), jnp.float32),
        ],
        compiler_params=pltpu.CompilerParams(
            dimension_semantics=("parallel",),
        ),
    )(x_crane, x_pile, maskT, Wc0, bc0, Wc1, bc1, Wp0, bp0, Wp1, bp1,
      A0c, A0p, A0b, A1, A1b, A2t, A2b)

    act, lp, val = pl.pallas_call(
        _fin_body,
        grid=(1,),
        in_specs=[
            full((NP, NC)),
            full((NBLK, D)),
            full((NC, D)),
            full((D, E)), full((1, E)), full((E, E)), full((1, E)),
            full((2 * E, 2 * E)), full((1, 2 * E)),
            full((2 * E, 2 * E)), full((1, 2 * E)),
            full((1, 2 * E)), full((1, 1)),
        ],
        out_specs=[
            pl.BlockSpec(memory_space=pltpu.SMEM),
            pl.BlockSpec(memory_space=pltpu.SMEM),
            pl.BlockSpec(memory_space=pltpu.SMEM),
        ],
        out_shape=[
            jax.ShapeDtypeStruct((1, 1), jnp.int32),
            jax.ShapeDtypeStruct((1, 1), jnp.float32),
            jax.ShapeDtypeStruct((1, 1), jnp.float32),
        ],
    )(lg, hps.reshape(NBLK, D), x_crane, Wc0, bc0, Wc1, bc1,
      C0, C0b, C1, C1b, C2t, C2b)
    return act[0, 0], lp[0, 0], val[0, 0]


def kernel(x_crane, x_pile, mask, crane_id,
           Wc0, bc0, Wc1, bc1, Wp0, bp0, Wp1, bp1,
           A0, A0b, A1, A1b, A2, A2b,
           C0, C0b, C1, C1b, C2, C2b):
    del crane_id
    row = lambda b: b.reshape(1, -1)
    return _run(
        x_crane, x_pile, mask.T.astype(jnp.float32),
        Wc0, row(bc0), Wc1, row(bc1), Wp0, row(bp0), Wp1, row(bp1),
        A0[:D], A0[D:], row(A0b), A1, row(A1b), A2.T, row(A2b),
        C0, row(C0b), C1, row(C1b), C2.T, row(C2b))


# speedup vs baseline: 1.1074x; 1.1074x over previous
import jax
import jax.numpy as jnp
from jax.experimental import pallas as pl
from jax.experimental.pallas import tpu as pltpu

NC, NP, D, E = 32, 2048, 128, 128
PB = 256
NBLK = NP // PB
NROW = NP * NC // 128
BROW = PB * NC // 128
NEG = -1e30


def _elu(x):
    return jnp.where(x > 0, x, jnp.exp(x) - 1.0)


def _body(xc_ref, xp_ref, mb_ref,
          Wc0_ref, bc0_ref, Wc1_ref, bc1_ref,
          Wp0_ref, bp0_ref, Wp1_ref, bp1_ref,
          A0c_ref, A0p_ref, A0b_ref, A1_ref, A1b_ref, A2_ref,
          C0_ref, C0b_ref, C1_ref, C1b_ref, C2t_ref, C2b_ref,
          act_ref, lp_ref, val_ref,
          U_scr, hcp_scr, hps_scr, lg_scr):
    i = pl.program_id(0)

    @pl.when(i == 0)
    def _init():
        hc = _elu(jnp.dot(xc_ref[:, :], Wc0_ref[:, :],
                          preferred_element_type=jnp.float32) + bc0_ref[:, :])
        hc = _elu(jnp.dot(hc, Wc1_ref[:, :],
                          preferred_element_type=jnp.float32) + bc1_ref[:, :])
        U_scr[:, :] = jnp.dot(hc, A0c_ref[:, :],
                              preferred_element_type=jnp.float32) + A0b_ref[:, :]
        hcp_scr[:, :] = jnp.mean(hc, axis=0, keepdims=True)
        hps_scr[:, :] = jnp.zeros((1, D), jnp.float32)

    hp = _elu(jnp.dot(xp_ref[:, :], Wp0_ref[:, :],
                      preferred_element_type=jnp.float32) + bp0_ref[:, :])
    hp = _elu(jnp.dot(hp, Wp1_ref[:, :],
                      preferred_element_type=jnp.float32) + bp1_ref[:, :])
    hps_scr[:, :] += jnp.sum(hp, axis=0, keepdims=True)

    V = jnp.dot(hp, A0p_ref[:, :], preferred_element_type=jnp.float32)
    ha = _elu(V[:, None, :] + U_scr[:, :][None, :, :])
    ha = ha.reshape(PB * NC, 2 * E)
    ha = _elu(jnp.dot(ha, A1_ref[:, :],
                      preferred_element_type=jnp.float32) + A1b_ref[:, :])
    lg = jnp.dot(ha, A2_ref[:, :], preferred_element_type=jnp.float32)
    lg_scr[pl.ds(i * BROW, BROW), :] = lg.reshape(BROW, 128)

    @pl.when(i == NBLK - 1)
    def _fin():
        full = lg_scr[:, :] + mb_ref[:, :]
        M = jnp.max(full)
        S = jnp.sum(jnp.exp(full - M))
        ridx = jax.lax.broadcasted_iota(jnp.int32, (NROW, 128), 0)
        lidx = jax.lax.broadcasted_iota(jnp.int32, (NROW, 128), 1)
        flat = ridx * 128 + lidx
        act_ref[0, 0] = jnp.min(jnp.where(full >= M, flat,
                                          jnp.int32(2147483647)))
        lp_ref[0, 0] = -jnp.log(S)
        hpool = jnp.concatenate([hcp_scr[:, :],
                                 hps_scr[:, :] * (1.0 / NP)], axis=1)
        hv = _elu(jnp.dot(hpool, C0_ref[:, :],
                          preferred_element_type=jnp.float32) + C0b_ref[:, :])
        hv = _elu(jnp.dot(hv, C1_ref[:, :],
                          preferred_element_type=jnp.float32) + C1b_ref[:, :])
        val_ref[0, 0] = jnp.sum(hv * C2t_ref[:, :]) + C2b_ref[0, 0]


@jax.jit
def _run(x_crane, x_pile, mbias, Wc0, bc0, Wc1, bc1, Wp0, bp0, Wp1, bp1,
         A0c, A0p, A0b, A1, A1b, A2, C0, C0b, C1, C1b, C2t, C2b):
    full = lambda shape: pl.BlockSpec(shape, lambda i: (0,) * len(shape))
    act, lp, val = pl.pallas_call(
        _body,
        grid=(NBLK,),
        in_specs=[
            full((NC, D)),
            pl.BlockSpec((PB, D), lambda i: (i, 0)),
            full((NROW, 128)),
            full((D, E)), full((1, E)), full((E, E)), full((1, E)),
            full((D, E)), full((1, E)), full((E, E)), full((1, E)),
            full((D, 2 * E)), full((D, 2 * E)), full((1, 2 * E)),
            full((2 * E, 2 * E)), full((1, 2 * E)),
            full((2 * E, 1)),
            full((2 * E, 2 * E)), full((1, 2 * E)),
            full((2 * E, 2 * E)), full((1, 2 * E)),
            full((1, 2 * E)), full((1, 1)),
        ],
        out_specs=[
            pl.BlockSpec(memory_space=pltpu.SMEM),
            pl.BlockSpec(memory_space=pltpu.SMEM),
            pl.BlockSpec(memory_space=pltpu.SMEM),
        ],
        out_shape=[
            jax.ShapeDtypeStruct((1, 1), jnp.int32),
            jax.ShapeDtypeStruct((1, 1), jnp.float32),
            jax.ShapeDtypeStruct((1, 1), jnp.float32),
        ],
        scratch_shapes=[
            pltpu.VMEM((NC, 2 * E), jnp.float32),
            pltpu.VMEM((1, D), jnp.float32),
            pltpu.VMEM((1, D), jnp.float32),
            pltpu.VMEM((NROW, 128), jnp.float32),
        ],
        compiler_params=pltpu.CompilerParams(
            dimension_semantics=("arbitrary",),
        ),
    )(x_crane, x_pile, mbias, Wc0, bc0, Wc1, bc1, Wp0, bp0, Wp1, bp1,
      A0c, A0p, A0b, A1, A1b, A2, C0, C0b, C1, C1b, C2t, C2b)
    return act[0, 0], lp[0, 0], val[0, 0]


def kernel(x_crane, x_pile, mask, crane_id,
           Wc0, bc0, Wc1, bc1, Wp0, bp0, Wp1, bp1,
           A0, A0b, A1, A1b, A2, A2b,
           C0, C0b, C1, C1b, C2, C2b):
    del crane_id, A2b
    row = lambda b: b.reshape(1, -1)
    mbias = jnp.where(mask.T.reshape(NROW, 128), 0.0, NEG).astype(jnp.float32)
    return _run(
        x_crane, x_pile, mbias,
        Wc0, row(bc0), Wc1, row(bc1), Wp0, row(bp0), Wp1, row(bp1),
        A0[:D], A0[D:], row(A0b), A1, row(A1b), A2,
        C0, row(C0b), C1, row(C1b), C2.T, row(C2b))


# speedup vs baseline: 1.1939x; 1.0781x over previous
import jax
import jax.numpy as jnp
from jax.experimental import pallas as pl
from jax.experimental.pallas import tpu as pltpu

NC, NP, D, E = 32, 2048, 128, 128
PB = 512
NBLK = NP // PB
NROW = NP * NC // 128
BROW = PB * NC // 128
NEG = -1e30


def _elu(x):
    return jnp.where(x > 0, x, jnp.exp(x) - 1.0)


def _body(xc_ref, xp_ref, mb_ref,
          Wc0_ref, bc0_ref, Wc1_ref, bc1_ref,
          Wp0_ref, bp0_ref, Wp1_ref, bp1_ref,
          A0c_ref, A0p_ref, A0b_ref, A1_ref, A1b_ref, A2_ref,
          C0_ref, C0b_ref, C1_ref, C1b_ref, C2t_ref, C2b_ref,
          act_ref, lp_ref, val_ref,
          U_scr, hcp_scr, hps_scr, lg_scr):
    i = pl.program_id(0)

    @pl.when(i == 0)
    def _init():
        hc = _elu(jnp.dot(xc_ref[:, :], Wc0_ref[:, :],
                          preferred_element_type=jnp.float32) + bc0_ref[:, :])
        hc = _elu(jnp.dot(hc, Wc1_ref[:, :],
                          preferred_element_type=jnp.float32) + bc1_ref[:, :])
        U_scr[:, :] = jnp.dot(hc, A0c_ref[:, :],
                              preferred_element_type=jnp.float32) + A0b_ref[:, :]
        hcp_scr[:, :] = jnp.mean(hc, axis=0, keepdims=True)
        hps_scr[:, :] = jnp.zeros((1, D), jnp.float32)

    hp = _elu(jnp.dot(xp_ref[:, :], Wp0_ref[:, :],
                      preferred_element_type=jnp.float32) + bp0_ref[:, :])
    hp = _elu(jnp.dot(hp, Wp1_ref[:, :],
                      preferred_element_type=jnp.float32) + bp1_ref[:, :])
    hps_scr[:, :] += jnp.sum(hp, axis=0, keepdims=True)

    V = jnp.dot(hp, A0p_ref[:, :], preferred_element_type=jnp.float32)
    ha = _elu(V[:, None, :] + U_scr[:, :][None, :, :])
    ha = ha.reshape(PB * NC, 2 * E)
    ha = _elu(jnp.dot(ha, A1_ref[:, :],
                      preferred_element_type=jnp.float32) + A1b_ref[:, :])
    lg = jnp.dot(ha, A2_ref[:, :], preferred_element_type=jnp.float32)
    lg_scr[pl.ds(i * BROW, BROW), :] = lg.reshape(BROW, 128)

    @pl.when(i == NBLK - 1)
    def _fin():
        full = lg_scr[:, :] + mb_ref[:, :]
        M = jnp.max(full)
        S = jnp.sum(jnp.exp(full - M))
        ridx = jax.lax.broadcasted_iota(jnp.int32, (NROW, 128), 0)
        lidx = jax.lax.broadcasted_iota(jnp.int32, (NROW, 128), 1)
        flat = ridx * 128 + lidx
        act_ref[0, 0] = jnp.min(jnp.where(full >= M, flat,
                                          jnp.int32(2147483647)))
        lp_ref[0, 0] = -jnp.log(S)
        hpool = jnp.concatenate([hcp_scr[:, :],
                                 hps_scr[:, :] * (1.0 / NP)], axis=1)
        hv = _elu(jnp.dot(hpool, C0_ref[:, :],
                          preferred_element_type=jnp.float32) + C0b_ref[:, :])
        hv = _elu(jnp.dot(hv, C1_ref[:, :],
                          preferred_element_type=jnp.float32) + C1b_ref[:, :])
        val_ref[0, 0] = jnp.sum(hv * C2t_ref[:, :]) + C2b_ref[0, 0]


@jax.jit
def _run(x_crane, x_pile, mbias, Wc0, bc0, Wc1, bc1, Wp0, bp0, Wp1, bp1,
         A0c, A0p, A0b, A1, A1b, A2, C0, C0b, C1, C1b, C2t, C2b):
    full = lambda shape: pl.BlockSpec(shape, lambda i: (0,) * len(shape))
    act, lp, val = pl.pallas_call(
        _body,
        grid=(NBLK,),
        in_specs=[
            full((NC, D)),
            pl.BlockSpec((PB, D), lambda i: (i, 0)),
            full((NROW, 128)),
            full((D, E)), full((1, E)), full((E, E)), full((1, E)),
            full((D, E)), full((1, E)), full((E, E)), full((1, E)),
            full((D, 2 * E)), full((D, 2 * E)), full((1, 2 * E)),
            full((2 * E, 2 * E)), full((1, 2 * E)),
            full((2 * E, 1)),
            full((2 * E, 2 * E)), full((1, 2 * E)),
            full((2 * E, 2 * E)), full((1, 2 * E)),
            full((1, 2 * E)), full((1, 1)),
        ],
        out_specs=[
            pl.BlockSpec(memory_space=pltpu.SMEM),
            pl.BlockSpec(memory_space=pltpu.SMEM),
            pl.BlockSpec(memory_space=pltpu.SMEM),
        ],
        out_shape=[
            jax.ShapeDtypeStruct((1, 1), jnp.int32),
            jax.ShapeDtypeStruct((1, 1), jnp.float32),
            jax.ShapeDtypeStruct((1, 1), jnp.float32),
        ],
        scratch_shapes=[
            pltpu.VMEM((NC, 2 * E), jnp.float32),
            pltpu.VMEM((1, D), jnp.float32),
            pltpu.VMEM((1, D), jnp.float32),
            pltpu.VMEM((NROW, 128), jnp.float32),
        ],
        compiler_params=pltpu.CompilerParams(
            dimension_semantics=("arbitrary",),
        ),
    )(x_crane, x_pile, mbias, Wc0, bc0, Wc1, bc1, Wp0, bp0, Wp1, bp1,
      A0c, A0p, A0b, A1, A1b, A2, C0, C0b, C1, C1b, C2t, C2b)
    return act[0, 0], lp[0, 0], val[0, 0]


def kernel(x_crane, x_pile, mask, crane_id,
           Wc0, bc0, Wc1, bc1, Wp0, bp0, Wp1, bp1,
           A0, A0b, A1, A1b, A2, A2b,
           C0, C0b, C1, C1b, C2, C2b):
    del crane_id, A2b
    row = lambda b: b.reshape(1, -1)
    mbias = jnp.where(mask.T.reshape(NROW, 128), 0.0, NEG).astype(jnp.float32)
    return _run(
        x_crane, x_pile, mbias,
        Wc0, row(bc0), Wc1, row(bc1), Wp0, row(bp0), Wp1, row(bp1),
        A0[:D], A0[D:], row(A0b), A1, row(A1b), A2,
        C0, row(C0b), C1, row(C1b), C2.T, row(C2b))


# speedup vs baseline: 1.2176x; 1.0199x over previous
import jax
import jax.numpy as jnp
from jax.experimental import pallas as pl
from jax.experimental.pallas import tpu as pltpu

NC, NP, D, E = 32, 2048, 128, 128
PB = 1024
NBLK = NP // PB
NROW = NP * NC // 128
BROW = PB * NC // 128
NEG = -1e30


def _elu(x):
    return jnp.where(x > 0, x, jnp.exp(x) - 1.0)


def _body(xc_ref, xp_ref, mb_ref,
          Wc0_ref, bc0_ref, Wc1_ref, bc1_ref,
          Wp0_ref, bp0_ref, Wp1_ref, bp1_ref,
          A0c_ref, A0p_ref, A0b_ref, A1_ref, A1b_ref, A2_ref,
          C0_ref, C0b_ref, C1_ref, C1b_ref, C2t_ref, C2b_ref,
          act_ref, lp_ref, val_ref,
          U_scr, hcp_scr, hps_scr, lg_scr):
    i = pl.program_id(0)

    @pl.when(i == 0)
    def _init():
        hc = _elu(jnp.dot(xc_ref[:, :], Wc0_ref[:, :],
                          preferred_element_type=jnp.float32) + bc0_ref[:, :])
        hc = _elu(jnp.dot(hc, Wc1_ref[:, :],
                          preferred_element_type=jnp.float32) + bc1_ref[:, :])
        U_scr[:, :] = jnp.dot(hc, A0c_ref[:, :],
                              preferred_element_type=jnp.float32) + A0b_ref[:, :]
        hcp_scr[:, :] = jnp.mean(hc, axis=0, keepdims=True)
        hps_scr[:, :] = jnp.zeros((1, D), jnp.float32)

    hp = _elu(jnp.dot(xp_ref[:, :], Wp0_ref[:, :],
                      preferred_element_type=jnp.float32) + bp0_ref[:, :])
    hp = _elu(jnp.dot(hp, Wp1_ref[:, :],
                      preferred_element_type=jnp.float32) + bp1_ref[:, :])
    hps_scr[:, :] += jnp.sum(hp, axis=0, keepdims=True)

    V = jnp.dot(hp, A0p_ref[:, :], preferred_element_type=jnp.float32)
    ha = _elu(V[:, None, :] + U_scr[:, :][None, :, :])
    ha = ha.reshape(PB * NC, 2 * E)
    ha = _elu(jnp.dot(ha, A1_ref[:, :],
                      preferred_element_type=jnp.float32) + A1b_ref[:, :])
    lg = jnp.dot(ha, A2_ref[:, :], preferred_element_type=jnp.float32)
    lg_scr[pl.ds(i * BROW, BROW), :] = lg.reshape(BROW, 128)

    @pl.when(i == NBLK - 1)
    def _fin():
        full = lg_scr[:, :] + mb_ref[:, :]
        M = jnp.max(full)
        S = jnp.sum(jnp.exp(full - M))
        ridx = jax.lax.broadcasted_iota(jnp.int32, (NROW, 128), 0)
        lidx = jax.lax.broadcasted_iota(jnp.int32, (NROW, 128), 1)
        flat = ridx * 128 + lidx
        act_ref[0, 0] = jnp.min(jnp.where(full >= M, flat,
                                          jnp.int32(2147483647)))
        lp_ref[0, 0] = -jnp.log(S)
        hpool = jnp.concatenate([hcp_scr[:, :],
                                 hps_scr[:, :] * (1.0 / NP)], axis=1)
        hv = _elu(jnp.dot(hpool, C0_ref[:, :],
                          preferred_element_type=jnp.float32) + C0b_ref[:, :])
        hv = _elu(jnp.dot(hv, C1_ref[:, :],
                          preferred_element_type=jnp.float32) + C1b_ref[:, :])
        val_ref[0, 0] = jnp.sum(hv * C2t_ref[:, :]) + C2b_ref[0, 0]


@jax.jit
def _run(x_crane, x_pile, mbias, Wc0, bc0, Wc1, bc1, Wp0, bp0, Wp1, bp1,
         A0c, A0p, A0b, A1, A1b, A2, C0, C0b, C1, C1b, C2t, C2b):
    full = lambda shape: pl.BlockSpec(shape, lambda i: (0,) * len(shape))
    act, lp, val = pl.pallas_call(
        _body,
        grid=(NBLK,),
        in_specs=[
            full((NC, D)),
            pl.BlockSpec((PB, D), lambda i: (i, 0)),
            full((NROW, 128)),
            full((D, E)), full((1, E)), full((E, E)), full((1, E)),
            full((D, E)), full((1, E)), full((E, E)), full((1, E)),
            full((D, 2 * E)), full((D, 2 * E)), full((1, 2 * E)),
            full((2 * E, 2 * E)), full((1, 2 * E)),
            full((2 * E, 1)),
            full((2 * E, 2 * E)), full((1, 2 * E)),
            full((2 * E, 2 * E)), full((1, 2 * E)),
            full((1, 2 * E)), full((1, 1)),
        ],
        out_specs=[
            pl.BlockSpec(memory_space=pltpu.SMEM),
            pl.BlockSpec(memory_space=pltpu.SMEM),
            pl.BlockSpec(memory_space=pltpu.SMEM),
        ],
        out_shape=[
            jax.ShapeDtypeStruct((1, 1), jnp.int32),
            jax.ShapeDtypeStruct((1, 1), jnp.float32),
            jax.ShapeDtypeStruct((1, 1), jnp.float32),
        ],
        scratch_shapes=[
            pltpu.VMEM((NC, 2 * E), jnp.float32),
            pltpu.VMEM((1, D), jnp.float32),
            pltpu.VMEM((1, D), jnp.float32),
            pltpu.VMEM((NROW, 128), jnp.float32),
        ],
        compiler_params=pltpu.CompilerParams(
            dimension_semantics=("arbitrary",),
        ),
    )(x_crane, x_pile, mbias, Wc0, bc0, Wc1, bc1, Wp0, bp0, Wp1, bp1,
      A0c, A0p, A0b, A1, A1b, A2, C0, C0b, C1, C1b, C2t, C2b)
    return act[0, 0], lp[0, 0], val[0, 0]


def kernel(x_crane, x_pile, mask, crane_id,
           Wc0, bc0, Wc1, bc1, Wp0, bp0, Wp1, bp1,
           A0, A0b, A1, A1b, A2, A2b,
           C0, C0b, C1, C1b, C2, C2b):
    del crane_id, A2b
    row = lambda b: b.reshape(1, -1)
    mbias = jnp.where(mask.T.reshape(NROW, 128), 0.0, NEG).astype(jnp.float32)
    return _run(
        x_crane, x_pile, mbias,
        Wc0, row(bc0), Wc1, row(bc1), Wp0, row(bp0), Wp1, row(bp1),
        A0[:D], A0[D:], row(A0b), A1, row(A1b), A2,
        C0, row(C0b), C1, row(C1b), C2.T, row(C2b))


# speedup vs baseline: 1.2255x; 1.0065x over previous
import jax
import jax.numpy as jnp
from jax.experimental import pallas as pl
from jax.experimental.pallas import tpu as pltpu

NC, NP, D, E = 32, 2048, 128, 128
PB = 2048
NBLK = NP // PB
NROW = NP * NC // 128
BROW = PB * NC // 128
NEG = -1e30


def _elu(x):
    return jnp.where(x > 0, x, jnp.exp(x) - 1.0)


def _body(xc_ref, xp_ref, mb_ref,
          Wc0_ref, bc0_ref, Wc1_ref, bc1_ref,
          Wp0_ref, bp0_ref, Wp1_ref, bp1_ref,
          A0c_ref, A0p_ref, A0b_ref, A1_ref, A1b_ref, A2_ref,
          C0_ref, C0b_ref, C1_ref, C1b_ref, C2t_ref, C2b_ref,
          act_ref, lp_ref, val_ref,
          U_scr, hcp_scr, hps_scr, lg_scr):
    i = pl.program_id(0)

    @pl.when(i == 0)
    def _init():
        hc = _elu(jnp.dot(xc_ref[:, :], Wc0_ref[:, :],
                          preferred_element_type=jnp.float32) + bc0_ref[:, :])
        hc = _elu(jnp.dot(hc, Wc1_ref[:, :],
                          preferred_element_type=jnp.float32) + bc1_ref[:, :])
        U_scr[:, :] = jnp.dot(hc, A0c_ref[:, :],
                              preferred_element_type=jnp.float32) + A0b_ref[:, :]
        hcp_scr[:, :] = jnp.mean(hc, axis=0, keepdims=True)
        hps_scr[:, :] = jnp.zeros((1, D), jnp.float32)

    hp = _elu(jnp.dot(xp_ref[:, :], Wp0_ref[:, :],
                      preferred_element_type=jnp.float32) + bp0_ref[:, :])
    hp = _elu(jnp.dot(hp, Wp1_ref[:, :],
                      preferred_element_type=jnp.float32) + bp1_ref[:, :])
    hps_scr[:, :] += jnp.sum(hp, axis=0, keepdims=True)

    V = jnp.dot(hp, A0p_ref[:, :], preferred_element_type=jnp.float32)
    ha = _elu(V[:, None, :] + U_scr[:, :][None, :, :])
    ha = ha.reshape(PB * NC, 2 * E)
    ha = _elu(jnp.dot(ha, A1_ref[:, :],
                      preferred_element_type=jnp.float32) + A1b_ref[:, :])
    lg = jnp.dot(ha, A2_ref[:, :], preferred_element_type=jnp.float32)
    lg_scr[pl.ds(i * BROW, BROW), :] = lg.reshape(BROW, 128)

    @pl.when(i == NBLK - 1)
    def _fin():
        full = lg_scr[:, :] + mb_ref[:, :]
        M = jnp.max(full)
        S = jnp.sum(jnp.exp(full - M))
        ridx = jax.lax.broadcasted_iota(jnp.int32, (NROW, 128), 0)
        lidx = jax.lax.broadcasted_iota(jnp.int32, (NROW, 128), 1)
        flat = ridx * 128 + lidx
        act_ref[0, 0] = jnp.min(jnp.where(full >= M, flat,
                                          jnp.int32(2147483647)))
        lp_ref[0, 0] = -jnp.log(S)
        hpool = jnp.concatenate([hcp_scr[:, :],
                                 hps_scr[:, :] * (1.0 / NP)], axis=1)
        hv = _elu(jnp.dot(hpool, C0_ref[:, :],
                          preferred_element_type=jnp.float32) + C0b_ref[:, :])
        hv = _elu(jnp.dot(hv, C1_ref[:, :],
                          preferred_element_type=jnp.float32) + C1b_ref[:, :])
        val_ref[0, 0] = jnp.sum(hv * C2t_ref[:, :]) + C2b_ref[0, 0]


@jax.jit
def _run(x_crane, x_pile, mbias, Wc0, bc0, Wc1, bc1, Wp0, bp0, Wp1, bp1,
         A0c, A0p, A0b, A1, A1b, A2, C0, C0b, C1, C1b, C2t, C2b):
    full = lambda shape: pl.BlockSpec(shape, lambda i: (0,) * len(shape))
    act, lp, val = pl.pallas_call(
        _body,
        grid=(NBLK,),
        in_specs=[
            full((NC, D)),
            pl.BlockSpec((PB, D), lambda i: (i, 0)),
            full((NROW, 128)),
            full((D, E)), full((1, E)), full((E, E)), full((1, E)),
            full((D, E)), full((1, E)), full((E, E)), full((1, E)),
            full((D, 2 * E)), full((D, 2 * E)), full((1, 2 * E)),
            full((2 * E, 2 * E)), full((1, 2 * E)),
            full((2 * E, 1)),
            full((2 * E, 2 * E)), full((1, 2 * E)),
            full((2 * E, 2 * E)), full((1, 2 * E)),
            full((1, 2 * E)), full((1, 1)),
        ],
        out_specs=[
            pl.BlockSpec(memory_space=pltpu.SMEM),
            pl.BlockSpec(memory_space=pltpu.SMEM),
            pl.BlockSpec(memory_space=pltpu.SMEM),
        ],
        out_shape=[
            jax.ShapeDtypeStruct((1, 1), jnp.int32),
            jax.ShapeDtypeStruct((1, 1), jnp.float32),
            jax.ShapeDtypeStruct((1, 1), jnp.float32),
        ],
        scratch_shapes=[
            pltpu.VMEM((NC, 2 * E), jnp.float32),
            pltpu.VMEM((1, D), jnp.float32),
            pltpu.VMEM((1, D), jnp.float32),
            pltpu.VMEM((NROW, 128), jnp.float32),
        ],
        compiler_params=pltpu.CompilerParams(
            dimension_semantics=("arbitrary",),
        ),
    )(x_crane, x_pile, mbias, Wc0, bc0, Wc1, bc1, Wp0, bp0, Wp1, bp1,
      A0c, A0p, A0b, A1, A1b, A2, C0, C0b, C1, C1b, C2t, C2b)
    return act[0, 0], lp[0, 0], val[0, 0]


def kernel(x_crane, x_pile, mask, crane_id,
           Wc0, bc0, Wc1, bc1, Wp0, bp0, Wp1, bp1,
           A0, A0b, A1, A1b, A2, A2b,
           C0, C0b, C1, C1b, C2, C2b):
    del crane_id, A2b
    row = lambda b: b.reshape(1, -1)
    mbias = jnp.where(mask.T.reshape(NROW, 128), 0.0, NEG).astype(jnp.float32)
    return _run(
        x_crane, x_pile, mbias,
        Wc0, row(bc0), Wc1, row(bc1), Wp0, row(bp0), Wp1, row(bp1),
        A0[:D], A0[D:], row(A0b), A1, row(A1b), A2,
        C0, row(C0b), C1, row(C1b), C2.T, row(C2b))
